# Initial kernel scaffold; baseline (speedup 1.0000x reference)
import jax
import jax.numpy as jnp
from jax.experimental import pallas as pl


def kernel(tokens, embedding):
    raise NotImplementedError("write your pallas kernel here")


# speedup vs baseline: 3.9207x; 3.9207x over previous
import functools
import math

import jax
import jax.numpy as jnp
from jax import lax
from jax.experimental import pallas as pl
from jax.experimental.pallas import tpu as pltpu
from jax.experimental.pallas import tpu_sc as plsc


def _scale_body(x_ref, o_ref, *, scale):
    o_ref[...] = x_ref[...] * scale


def _scale_table(embedding, scale):
    V, D = embedding.shape
    blk = 2000
    assert V % blk == 0
    return pl.pallas_call(
        functools.partial(_scale_body, scale=scale),
        grid=(V // blk,),
        in_specs=[pl.BlockSpec((blk, D), lambda i: (i, 0))],
        out_specs=pl.BlockSpec((blk, D), lambda i: (i, 0)),
        out_shape=jax.ShapeDtypeStruct((V, D), jnp.float32),
    )(embedding)


def _make_gather(V, D, N):
    info = plsc.get_sparse_core_info()
    NC, NS = info.num_cores, info.num_subcores
    NW = NC * NS
    CH = 128
    NBUF = 4
    GROUP = CH * NBUF
    per_w = N // NW
    assert per_w % GROUP == 0
    n_chunks = per_w // CH
    n_groups = per_w // GROUP
    assert n_groups % 2 == 0 and n_groups >= 4

    mesh = plsc.VectorSubcoreMesh(core_axis_name="c", subcore_axis_name="s")

    @functools.partial(
        pl.kernel,
        out_type=jax.ShapeDtypeStruct((N, D), jnp.float32),
        mesh=mesh,
        scratch_types=[
            pltpu.VMEM((n_chunks, CH), jnp.int32),
            pltpu.VMEM((GROUP, D), jnp.float32),
            pltpu.VMEM((GROUP, D), jnp.float32),
            pltpu.SemaphoreType.DMA,
            pltpu.SemaphoreType.DMA,
            pltpu.SemaphoreType.DMA,
            pltpu.SemaphoreType.DMA,
        ],
        compiler_params=pltpu.CompilerParams(use_tc_tiling_on_sc=False),
    )
    def gather(table_hbm, idx_hbm, out_hbm, idx_v, buf0, buf1, gs0, gs1, ws0, ws1):
        wid = lax.axis_index("s") * NC + lax.axis_index("c")
        base = wid * per_w
        pltpu.sync_copy(idx_hbm.at[wid], idx_v)

        def fire_gathers(g, buf, gsem):
            for b in range(NBUF):
                j = g * NBUF + b
                pltpu.async_copy(
                    table_hbm.at[idx_v.at[j]], buf.at[pl.ds(b * CH, CH)], gsem
                )

        def drain_gathers(buf, gsem):
            pltpu.make_async_copy(table_hbm.at[pl.ds(0, GROUP)], buf, gsem).wait()

        def fire_write(g, buf, wsem):
            pltpu.async_copy(buf, out_hbm.at[pl.ds(base + g * GROUP, GROUP)], wsem)

        def drain_write(buf, wsem):
            pltpu.make_async_copy(buf, out_hbm.at[pl.ds(base, GROUP)], wsem).wait()

        fire_gathers(0, buf0, gs0)
        drain_gathers(buf0, gs0)
        fire_write(0, buf0, ws0)
        fire_gathers(1, buf1, gs1)

        def body(p, carry):
            g = 1 + 2 * p
            drain_gathers(buf1, gs1)
            fire_write(g, buf1, ws1)
            drain_write(buf0, ws0)
            fire_gathers(g + 1, buf0, gs0)
            g2 = g + 1
            drain_gathers(buf0, gs0)
            fire_write(g2, buf0, ws0)
            drain_write(buf1, ws1)
            fire_gathers(g2 + 1, buf1, gs1)
            return carry

        lax.fori_loop(0, (n_groups - 2) // 2, body, 0)

        drain_gathers(buf1, gs1)
        fire_write(n_groups - 1, buf1, ws1)
        drain_write(buf0, ws0)
        drain_write(buf1, ws1)

    return gather, NW, n_chunks, CH


def kernel(tokens, embedding):
    V, D = embedding.shape
    S, B = tokens.shape
    N = S * B
    scale = float(math.sqrt(D))

    scaled = _scale_table(embedding, scale)

    gather, NW, n_chunks, CH = _make_gather(V, D, N)
    idx3 = tokens.reshape(NW, n_chunks, CH).astype(jnp.int32)
    out = gather(scaled, idx3)
    return out.reshape(S, B, D)


# speedup vs baseline: 4.1673x; 1.0629x over previous
import functools
import math

import jax
import jax.numpy as jnp
from jax import lax
from jax.experimental import pallas as pl
from jax.experimental.pallas import tpu as pltpu
from jax.experimental.pallas import tpu_sc as plsc


def _make_gather(V, D, N, scale):
    info = plsc.get_sparse_core_info()
    NC, NS, L = info.num_cores, info.num_subcores, info.num_lanes
    NW = NC * NS
    CH = 128
    NBUF = 4
    GROUP = CH * NBUF
    UNROLL = 8
    per_w = N // NW
    assert per_w % GROUP == 0 and D % L == 0 and GROUP % UNROLL == 0
    n_chunks = per_w // CH
    n_groups = per_w // GROUP
    assert n_groups % 2 == 0 and n_groups >= 4

    mesh = plsc.VectorSubcoreMesh(core_axis_name="c", subcore_axis_name="s")

    @functools.partial(
        pl.kernel,
        out_type=jax.ShapeDtypeStruct((N, D), jnp.float32),
        mesh=mesh,
        scratch_types=[
            pltpu.VMEM((n_chunks, CH), jnp.int32),
            pltpu.VMEM((GROUP, D), jnp.float32),
            pltpu.VMEM((GROUP, D), jnp.float32),
            pltpu.SemaphoreType.DMA,
            pltpu.SemaphoreType.DMA,
            pltpu.SemaphoreType.DMA,
            pltpu.SemaphoreType.DMA,
        ],
        compiler_params=pltpu.CompilerParams(use_tc_tiling_on_sc=False),
    )
    def gather(table_hbm, idx_hbm, out_hbm, idx_v, buf0, buf1, gs0, gs1, ws0, ws1):
        wid = lax.axis_index("s") * NC + lax.axis_index("c")
        base = wid * per_w
        pltpu.sync_copy(idx_hbm.at[wid], idx_v)

        def fire_gathers(g, buf, gsem):
            for b in range(NBUF):
                j = g * NBUF + b
                pltpu.async_copy(
                    table_hbm.at[idx_v.at[j]], buf.at[pl.ds(b * CH, CH)], gsem
                )

        def drain_gathers(buf, gsem):
            pltpu.make_async_copy(table_hbm.at[pl.ds(0, GROUP)], buf, gsem).wait()

        def scale_bank(buf):
            def sbody(i, carry):
                r0 = i * UNROLL
                for dr in range(UNROLL):
                    for c in range(D // L):
                        sl = pl.ds(c * L, L)
                        buf[r0 + dr, sl] = buf[r0 + dr, sl] * scale
                return carry

            lax.fori_loop(0, GROUP // UNROLL, sbody, 0)

        def fire_write(g, buf, wsem):
            pltpu.async_copy(buf, out_hbm.at[pl.ds(base + g * GROUP, GROUP)], wsem)

        def drain_write(buf, wsem):
            pltpu.make_async_copy(buf, out_hbm.at[pl.ds(base, GROUP)], wsem).wait()

        fire_gathers(0, buf0, gs0)
        fire_gathers(1, buf1, gs1)
        drain_gathers(buf0, gs0)
        scale_bank(buf0)
        fire_write(0, buf0, ws0)

        def body(p, carry):
            g = 1 + 2 * p
            drain_gathers(buf1, gs1)
            scale_bank(buf1)
            fire_write(g, buf1, ws1)
            drain_write(buf0, ws0)
            fire_gathers(g + 1, buf0, gs0)
            g2 = g + 1
            drain_gathers(buf0, gs0)
            scale_bank(buf0)
            fire_write(g2, buf0, ws0)
            drain_write(buf1, ws1)
            fire_gathers(g2 + 1, buf1, gs1)
            return carry

        lax.fori_loop(0, (n_groups - 2) // 2, body, 0)

        drain_gathers(buf1, gs1)
        scale_bank(buf1)
        fire_write(n_groups - 1, buf1, ws1)
        drain_write(buf0, ws0)
        drain_write(buf1, ws1)

    return gather, NW, n_chunks, CH


def kernel(tokens, embedding):
    V, D = embedding.shape
    S, B = tokens.shape
    N = S * B
    scale = float(math.sqrt(D))

    gather, NW, n_chunks, CH = _make_gather(V, D, N, scale)
    idx3 = tokens.reshape(NW, n_chunks, CH).astype(jnp.int32)
    out = gather(embedding, idx3)
    return out.reshape(S, B, D)
